# baseline (speedup 1.0000x reference)
import functools

import jax
import jax.numpy as jnp
from jax import lax
from jax.experimental import pallas as pl
from jax.experimental.pallas import tpu as pltpu
from jax.experimental.pallas import tpu_sc as plsc

DIM = 64
MARGIN = 6.0
NC, NS = 2, 16
NW = NC * NS
IW = 128
PB = 4096
HW = 2048
NCHUNK = 2


def _tc_pack_e2(re_t, im_t):
    n = re_t.shape[1]

    def body(re_ref, im_ref, e2_ref):
        e2_ref[...] = jnp.concatenate([re_ref[...].T, im_ref[...].T], axis=1)

    return pl.pallas_call(
        body,
        grid=(pl.cdiv(n, PB),),
        in_specs=[pl.BlockSpec((DIM, PB), lambda i: (0, i))] * 2,
        out_specs=pl.BlockSpec((PB, 2 * DIM), lambda i: (i, 0)),
        out_shape=jax.ShapeDtypeStruct((n, 2 * DIM), jnp.float32),
    )(re_t, im_t)


def _tc_pack_p2(ph_t):
    n = ph_t.shape[1]

    def body(ph_ref, p2_ref):
        p = ph_ref[...].T
        p2_ref[...] = jnp.concatenate([p, p], axis=1)

    return pl.pallas_call(
        body,
        grid=(pl.cdiv(n, PB),),
        in_specs=[pl.BlockSpec((DIM, PB), lambda i: (0, i))],
        out_specs=pl.BlockSpec((PB, 2 * DIM), lambda i: (i, 0)),
        out_shape=jax.ShapeDtypeStruct((n, 2 * DIM), jnp.float32),
    )(ph_t)


def _sc_gather(idx_list, table, total, ch):
    nrole = len(idx_list)
    b_per_w = ch * IW
    mesh = plsc.VectorSubcoreMesh(core_axis_name="c", subcore_axis_name="s")
    row_t = jax.ShapeDtypeStruct((total, 2 * DIM), jnp.float32)

    @functools.partial(
        pl.kernel, mesh=mesh,
        out_type=[row_t] * nrole,
        scratch_types=[pltpu.VMEM((ch, IW), jnp.int32)] * nrole
        + [pltpu.VMEM((2 * IW, 2 * DIM), jnp.float32)] * 2
        + [pltpu.SemaphoreType.DMA] * 2,
    )
    def k(*refs):
        idx_hbm = refs[:nrole]
        table_hbm = refs[nrole]
        outs = refs[nrole + 1:2 * nrole + 1]
        idx_v = refs[2 * nrole + 1:3 * nrole + 1]
        buf0, buf1, sem_g, sem_w = refs[3 * nrole + 1:]
        wid = lax.axis_index("s") * NC + lax.axis_index("c")
        base = wid * b_per_w
        for r in range(nrole):
            pltpu.sync_copy(idx_hbm[r].at[wid], idx_v[r])

        bufs = (buf0, buf1)
        writes = [None, None]
        step = 0
        for r in range(nrole):
            for h in range(ch // 2):
                b = step % 2
                if writes[b] is not None:
                    writes[b].wait()
                g0 = pltpu.async_copy(table_hbm.at[idx_v[r].at[2 * h]],
                                      bufs[b].at[pl.ds(0, IW)], sem_g)
                g1 = pltpu.async_copy(table_hbm.at[idx_v[r].at[2 * h + 1]],
                                      bufs[b].at[pl.ds(IW, IW)], sem_g)
                g0.wait()
                g1.wait()
                writes[b] = pltpu.async_copy(
                    bufs[b], outs[r].at[pl.ds(base + h * 2 * IW, 2 * IW)],
                    sem_w)
                step += 1
        writes[0].wait()
        writes[1].wait()

    return k(*idx_list, table)


def _tc_loss_sum(g_h, g_t, g_r, rows):
    g = rows // (2 * HW)

    sin_c = (0.9999999378197463, -0.16666621108235025, 0.008332791502704946,
             -0.00019817630987702638, 2.70883115859738e-06,
             -2.0698134650665168e-08)
    cos_c = (0.9999992107795053, -0.4999942133837966, 0.041659777806388416,
             -0.0013858789919373926, 2.4202941365944475e-05,
             -2.1972963820671154e-07)

    def body(gh, gt, gr, out):
        ones = jnp.ones((2 * DIM, 2 * DIM), jnp.float32)
        mask = lax.broadcasted_iota(jnp.int32, (2 * HW, 2 * DIM), 1) < DIM
        mrow = lax.broadcasted_iota(jnp.int32, (1, 2 * DIM), 1) < DIM
        coef = [jnp.where(mrow, c, s).astype(jnp.float32)
                for c, s in zip(cos_c, sin_c)]

        def swap(x):
            return jnp.roll(x, DIM, axis=1)

        a = gh[...]
        t = gt[...]
        r = gr[...]
        y = r * r
        p = coef[5]
        for k in (4, 3, 2, 1, 0):
            p = p * y + coef[k]
        cs = jnp.where(mask, p, p * r)
        u = a * cs
        v = a * swap(cs)
        dre2 = u - swap(u)
        dim2 = v + swap(v)
        rot = jnp.where(mask, dre2, dim2)
        diff = rot - t
        sq = diff * diff
        val = jnp.sqrt(sq + swap(sq) + 1e-9)
        mag = jax.lax.dot_general(
            val, ones, (((1,), (0,)), ((), ())),
            preferred_element_type=jnp.float32)
        ms = jnp.maximum(MARGIN + 0.5 * (mag[:HW] - mag[HW:]), 0.0)
        i = pl.program_id(0)

        @pl.when(i == 0)
        def _():
            out[...] = jnp.zeros((1, 1), jnp.float32)

        out[...] += (jnp.sum(ms) / (2 * DIM)).reshape(1, 1)

    spec = pl.BlockSpec((2 * HW, 2 * DIM), lambda i: (i, 0))
    return pl.pallas_call(
        body,
        grid=(g,),
        in_specs=[spec] * 3,
        out_specs=pl.BlockSpec((1, 1), lambda i: (0, 0)),
        out_shape=jax.ShapeDtypeStruct((1, 1), jnp.float32),
    )(g_h, g_t, g_r)


def kernel(positive_triples, negative_triples, entity_re, entity_im,
           relation_phase):
    batch = positive_triples.shape[0]
    cb = batch // NCHUNK
    rows = 2 * cb
    ch = rows // (NW * IW)
    nh = cb // HW
    pt = positive_triples.astype(jnp.int32)
    nt = negative_triples.astype(jnp.int32)

    def order(col_p, col_n):
        mixed = jnp.concatenate([col_p.reshape(nh, HW),
                                 col_n.reshape(nh, HW)], axis=1)
        return mixed.reshape(NW, ch, IW)

    idx = []
    for c in range(NCHUNK):
        sl = slice(c * cb, (c + 1) * cb)
        idx.append([order(pt[sl, k], nt[sl, k]) for k in range(3)])

    e2 = _tc_pack_e2(entity_re.T, entity_im.T)
    ht = [_sc_gather([idx[c][0], idx[c][2]], e2, rows, ch)
          for c in range(NCHUNK)]
    p2 = _tc_pack_p2(relation_phase.T)
    rr = [_sc_gather([idx[c][1]], p2, rows, ch) for c in range(NCHUNK)]
    parts = [_tc_loss_sum(ht[c][0], ht[c][1], rr[c][0], rows)
             for c in range(NCHUNK)]
    total = parts[0]
    for p in parts[1:]:
        total = total + p
    return total[0, 0] / batch


# speedup vs baseline: 1.0285x; 1.0285x over previous
import functools

import jax
import jax.numpy as jnp
from jax import lax
from jax.experimental import pallas as pl
from jax.experimental.pallas import tpu as pltpu
from jax.experimental.pallas import tpu_sc as plsc

DIM = 64
MARGIN = 6.0
NC, NS = 2, 16
NW = NC * NS
IW = 128
PB = 4096
HW = 1024
NCHUNK = 1


def _tc_pack_e2(re_t, im_t):
    n = re_t.shape[1]

    def body(re_ref, im_ref, e2_ref):
        e2_ref[...] = jnp.concatenate([re_ref[...].T, im_ref[...].T], axis=1)

    return pl.pallas_call(
        body,
        grid=(pl.cdiv(n, PB),),
        in_specs=[pl.BlockSpec((DIM, PB), lambda i: (0, i))] * 2,
        out_specs=pl.BlockSpec((PB, 2 * DIM), lambda i: (i, 0)),
        out_shape=jax.ShapeDtypeStruct((n, 2 * DIM), jnp.float32),
    )(re_t, im_t)


def _tc_pack_p2(ph_t):
    n = ph_t.shape[1]

    def body(ph_ref, p2_ref):
        p = ph_ref[...].T
        p2_ref[...] = jnp.concatenate([p, p], axis=1)

    return pl.pallas_call(
        body,
        grid=(pl.cdiv(n, PB),),
        in_specs=[pl.BlockSpec((DIM, PB), lambda i: (0, i))],
        out_specs=pl.BlockSpec((PB, 2 * DIM), lambda i: (i, 0)),
        out_shape=jax.ShapeDtypeStruct((n, 2 * DIM), jnp.float32),
    )(ph_t)


def _sc_gather(idx_list, table, total, ch):
    nrole = len(idx_list)
    b_per_w = ch * IW
    mesh = plsc.VectorSubcoreMesh(core_axis_name="c", subcore_axis_name="s")
    row_t = jax.ShapeDtypeStruct((total, 2 * DIM), jnp.float32)

    @functools.partial(
        pl.kernel, mesh=mesh,
        out_type=[row_t] * nrole,
        scratch_types=[pltpu.VMEM((ch, IW), jnp.int32)] * nrole
        + [pltpu.VMEM((2 * IW, 2 * DIM), jnp.float32)] * 2
        + [pltpu.SemaphoreType.DMA] * 2,
    )
    def k(*refs):
        idx_hbm = refs[:nrole]
        table_hbm = refs[nrole]
        outs = refs[nrole + 1:2 * nrole + 1]
        idx_v = refs[2 * nrole + 1:3 * nrole + 1]
        buf0, buf1, sem_g, sem_w = refs[3 * nrole + 1:]
        wid = lax.axis_index("s") * NC + lax.axis_index("c")
        base = wid * b_per_w
        for r in range(nrole):
            pltpu.sync_copy(idx_hbm[r].at[wid], idx_v[r])

        bufs = (buf0, buf1)
        writes = [None, None]
        step = 0
        for r in range(nrole):
            for h in range(ch // 2):
                b = step % 2
                if writes[b] is not None:
                    writes[b].wait()
                g0 = pltpu.async_copy(table_hbm.at[idx_v[r].at[2 * h]],
                                      bufs[b].at[pl.ds(0, IW)], sem_g)
                g1 = pltpu.async_copy(table_hbm.at[idx_v[r].at[2 * h + 1]],
                                      bufs[b].at[pl.ds(IW, IW)], sem_g)
                g0.wait()
                g1.wait()
                writes[b] = pltpu.async_copy(
                    bufs[b], outs[r].at[pl.ds(base + h * 2 * IW, 2 * IW)],
                    sem_w)
                step += 1
        writes[0].wait()
        writes[1].wait()

    return k(*idx_list, table)


def _tc_loss_sum(g_h, g_t, g_r, rows):
    g = rows // (2 * HW)

    sin_c = (0.9999999378197463, -0.16666621108235025, 0.008332791502704946,
             -0.00019817630987702638, 2.70883115859738e-06,
             -2.0698134650665168e-08)
    cos_c = (0.9999992107795053, -0.4999942133837966, 0.041659777806388416,
             -0.0013858789919373926, 2.4202941365944475e-05,
             -2.1972963820671154e-07)

    def body(gh, gt, gr, out):
        ones = jnp.ones((2 * DIM, 2 * DIM), jnp.float32)
        mask = lax.broadcasted_iota(jnp.int32, (2 * HW, 2 * DIM), 1) < DIM
        mrow = lax.broadcasted_iota(jnp.int32, (1, 2 * DIM), 1) < DIM
        coef = [jnp.where(mrow, c, s).astype(jnp.float32)
                for c, s in zip(cos_c, sin_c)]

        def swap(x):
            return jnp.roll(x, DIM, axis=1)

        a = gh[...]
        t = gt[...]
        r = gr[...]
        y = r * r
        p = coef[5]
        for k in (4, 3, 2, 1, 0):
            p = p * y + coef[k]
        cs = jnp.where(mask, p, p * r)
        u = a * cs
        v = a * swap(cs)
        dre2 = u - swap(u)
        dim2 = v + swap(v)
        rot = jnp.where(mask, dre2, dim2)
        diff = rot - t
        sq = diff * diff
        val = jnp.sqrt(sq + swap(sq) + 1e-9)
        mag = jax.lax.dot_general(
            val, ones, (((1,), (0,)), ((), ())),
            preferred_element_type=jnp.float32)
        ms = jnp.maximum(MARGIN + 0.5 * (mag[:HW] - mag[HW:]), 0.0)
        i = pl.program_id(0)

        @pl.when(i == 0)
        def _():
            out[...] = jnp.zeros((1, 1), jnp.float32)

        out[...] += (jnp.sum(ms) / (2 * DIM)).reshape(1, 1)

    spec = pl.BlockSpec((2 * HW, 2 * DIM), lambda i: (i, 0))
    return pl.pallas_call(
        body,
        grid=(g,),
        in_specs=[spec] * 3,
        out_specs=pl.BlockSpec((1, 1), lambda i: (0, 0)),
        out_shape=jax.ShapeDtypeStruct((1, 1), jnp.float32),
    )(g_h, g_t, g_r)


def kernel(positive_triples, negative_triples, entity_re, entity_im,
           relation_phase):
    batch = positive_triples.shape[0]
    cb = batch // NCHUNK
    rows = 2 * cb
    ch = rows // (NW * IW)
    nh = cb // HW
    pt = positive_triples.astype(jnp.int32)
    nt = negative_triples.astype(jnp.int32)

    def order(col_p, col_n):
        mixed = jnp.concatenate([col_p.reshape(nh, HW),
                                 col_n.reshape(nh, HW)], axis=1)
        return mixed.reshape(NW, ch, IW)

    idx = []
    for c in range(NCHUNK):
        sl = slice(c * cb, (c + 1) * cb)
        idx.append([order(pt[sl, k], nt[sl, k]) for k in range(3)])

    e2 = _tc_pack_e2(entity_re.T, entity_im.T)
    ht = [_sc_gather([idx[c][0], idx[c][2]], e2, rows, ch)
          for c in range(NCHUNK)]
    p2 = _tc_pack_p2(relation_phase.T)
    rr = [_sc_gather([idx[c][1]], p2, rows, ch) for c in range(NCHUNK)]
    parts = [_tc_loss_sum(ht[c][0], ht[c][1], rr[c][0], rows)
             for c in range(NCHUNK)]
    total = parts[0]
    for p in parts[1:]:
        total = total + p
    return total[0, 0] / batch


# speedup vs baseline: 1.0776x; 1.0477x over previous
import functools

import jax
import jax.numpy as jnp
from jax import lax
from jax.experimental import pallas as pl
from jax.experimental.pallas import tpu as pltpu
from jax.experimental.pallas import tpu_sc as plsc

DIM = 64
MARGIN = 6.0
NC, NS = 2, 16
NW = NC * NS
IW = 128
PB = 8192
HW = 2048
NCHUNK = 1


def _tc_pack_e2(re_t, im_t):
    n = re_t.shape[1]

    def body(re_ref, im_ref, e2_ref):
        e2_ref[...] = jnp.concatenate([re_ref[...].T, im_ref[...].T], axis=1)

    return pl.pallas_call(
        body,
        grid=(pl.cdiv(n, PB),),
        in_specs=[pl.BlockSpec((DIM, PB), lambda i: (0, i))] * 2,
        out_specs=pl.BlockSpec((PB, 2 * DIM), lambda i: (i, 0)),
        out_shape=jax.ShapeDtypeStruct((n, 2 * DIM), jnp.float32),
    )(re_t, im_t)


def _tc_pack_p2(ph_t):
    n = ph_t.shape[1]

    def body(ph_ref, p2_ref):
        p = ph_ref[...].T
        p2_ref[...] = jnp.concatenate([p, p], axis=1)

    return pl.pallas_call(
        body,
        grid=(pl.cdiv(n, PB),),
        in_specs=[pl.BlockSpec((DIM, PB), lambda i: (0, i))],
        out_specs=pl.BlockSpec((PB, 2 * DIM), lambda i: (i, 0)),
        out_shape=jax.ShapeDtypeStruct((n, 2 * DIM), jnp.float32),
    )(ph_t)


def _sc_gather(idx_list, table, total, ch):
    nrole = len(idx_list)
    b_per_w = ch * IW
    mesh = plsc.VectorSubcoreMesh(core_axis_name="c", subcore_axis_name="s")
    row_t = jax.ShapeDtypeStruct((total, 2 * DIM), jnp.float32)

    @functools.partial(
        pl.kernel, mesh=mesh,
        out_type=[row_t] * nrole,
        scratch_types=[pltpu.VMEM((ch, IW), jnp.int32)] * nrole
        + [pltpu.VMEM((2 * IW, 2 * DIM), jnp.float32)] * 2
        + [pltpu.SemaphoreType.DMA] * 2,
    )
    def k(*refs):
        idx_hbm = refs[:nrole]
        table_hbm = refs[nrole]
        outs = refs[nrole + 1:2 * nrole + 1]
        idx_v = refs[2 * nrole + 1:3 * nrole + 1]
        buf0, buf1, sem_g, sem_w = refs[3 * nrole + 1:]
        wid = lax.axis_index("s") * NC + lax.axis_index("c")
        base = wid * b_per_w
        for r in range(nrole):
            pltpu.sync_copy(idx_hbm[r].at[wid], idx_v[r])

        bufs = (buf0, buf1)
        writes = [None, None]
        step = 0
        for r in range(nrole):
            for h in range(ch // 2):
                b = step % 2
                if writes[b] is not None:
                    writes[b].wait()
                g0 = pltpu.async_copy(table_hbm.at[idx_v[r].at[2 * h]],
                                      bufs[b].at[pl.ds(0, IW)], sem_g)
                g1 = pltpu.async_copy(table_hbm.at[idx_v[r].at[2 * h + 1]],
                                      bufs[b].at[pl.ds(IW, IW)], sem_g)
                g0.wait()
                g1.wait()
                writes[b] = pltpu.async_copy(
                    bufs[b], outs[r].at[pl.ds(base + h * 2 * IW, 2 * IW)],
                    sem_w)
                step += 1
        writes[0].wait()
        writes[1].wait()

    return k(*idx_list, table)


def _tc_loss_sum(g_h, g_t, g_r, rows):
    g = rows // (2 * HW)

    sin_c = (0.9999999378197463, -0.16666621108235025, 0.008332791502704946,
             -0.00019817630987702638, 2.70883115859738e-06,
             -2.0698134650665168e-08)
    cos_c = (0.9999992107795053, -0.4999942133837966, 0.041659777806388416,
             -0.0013858789919373926, 2.4202941365944475e-05,
             -2.1972963820671154e-07)

    def body(gh, gt, gr, out):
        ones = jnp.ones((2 * DIM, 2 * DIM), jnp.float32)
        mask = lax.broadcasted_iota(jnp.int32, (2 * HW, 2 * DIM), 1) < DIM
        mrow = lax.broadcasted_iota(jnp.int32, (1, 2 * DIM), 1) < DIM
        coef = [jnp.where(mrow, c, s).astype(jnp.float32)
                for c, s in zip(cos_c, sin_c)]

        def swap(x):
            return jnp.roll(x, DIM, axis=1)

        a = gh[...]
        t = gt[...]
        r = gr[...]
        y = r * r
        p = coef[5]
        for k in (4, 3, 2, 1, 0):
            p = p * y + coef[k]
        cs = jnp.where(mask, p, p * r)
        u = a * cs
        v = a * swap(cs)
        dre2 = u - swap(u)
        dim2 = v + swap(v)
        rot = jnp.where(mask, dre2, dim2)
        diff = rot - t
        sq = diff * diff
        val = jnp.sqrt(sq + swap(sq) + 1e-9)
        mag = jax.lax.dot_general(
            val, ones, (((1,), (0,)), ((), ())),
            preferred_element_type=jnp.float32)
        ms = jnp.maximum(MARGIN + 0.5 * (mag[:HW] - mag[HW:]), 0.0)
        i = pl.program_id(0)

        @pl.when(i == 0)
        def _():
            out[...] = jnp.zeros((1, 1), jnp.float32)

        out[...] += (jnp.sum(ms) / (2 * DIM)).reshape(1, 1)

    spec = pl.BlockSpec((2 * HW, 2 * DIM), lambda i: (i, 0))
    return pl.pallas_call(
        body,
        grid=(g,),
        in_specs=[spec] * 3,
        out_specs=pl.BlockSpec((1, 1), lambda i: (0, 0)),
        out_shape=jax.ShapeDtypeStruct((1, 1), jnp.float32),
    )(g_h, g_t, g_r)


def kernel(positive_triples, negative_triples, entity_re, entity_im,
           relation_phase):
    batch = positive_triples.shape[0]
    cb = batch // NCHUNK
    rows = 2 * cb
    ch = rows // (NW * IW)
    nh = cb // HW
    pt = positive_triples.astype(jnp.int32)
    nt = negative_triples.astype(jnp.int32)

    def order(col_p, col_n):
        mixed = jnp.concatenate([col_p.reshape(nh, HW),
                                 col_n.reshape(nh, HW)], axis=1)
        return mixed.reshape(NW, ch, IW)

    idx = []
    for c in range(NCHUNK):
        sl = slice(c * cb, (c + 1) * cb)
        idx.append([order(pt[sl, k], nt[sl, k]) for k in range(3)])

    e2 = _tc_pack_e2(entity_re.T, entity_im.T)
    ht = [_sc_gather([idx[c][0], idx[c][2]], e2, rows, ch)
          for c in range(NCHUNK)]
    p2 = _tc_pack_p2(relation_phase.T)
    rr = [_sc_gather([idx[c][1]], p2, rows, ch) for c in range(NCHUNK)]
    parts = [_tc_loss_sum(ht[c][0], ht[c][1], rr[c][0], rows)
             for c in range(NCHUNK)]
    total = parts[0]
    for p in parts[1:]:
        total = total + p
    return total[0, 0] / batch


# speedup vs baseline: 1.0998x; 1.0207x over previous
import functools

import jax
import jax.numpy as jnp
from jax import lax
from jax.experimental import pallas as pl
from jax.experimental.pallas import tpu as pltpu
from jax.experimental.pallas import tpu_sc as plsc

DIM = 64
MARGIN = 6.0
NC, NS = 2, 16
NW = NC * NS
IW = 128
PB = 12800
HW = 2048
NCHUNK = 1


def _tc_pack_e2(re_t, im_t):
    n = re_t.shape[1]

    def body(re_ref, im_ref, e2_ref):
        e2_ref[...] = jnp.concatenate([re_ref[...].T, im_ref[...].T], axis=1)

    return pl.pallas_call(
        body,
        grid=(pl.cdiv(n, PB),),
        in_specs=[pl.BlockSpec((DIM, PB), lambda i: (0, i))] * 2,
        out_specs=pl.BlockSpec((PB, 2 * DIM), lambda i: (i, 0)),
        out_shape=jax.ShapeDtypeStruct((n, 2 * DIM), jnp.float32),
    )(re_t, im_t)


def _tc_pack_p2(ph_t):
    n = ph_t.shape[1]

    def body(ph_ref, p2_ref):
        p = ph_ref[...].T
        p2_ref[...] = jnp.concatenate([p, p], axis=1)

    return pl.pallas_call(
        body,
        grid=(pl.cdiv(n, PB),),
        in_specs=[pl.BlockSpec((DIM, PB), lambda i: (0, i))],
        out_specs=pl.BlockSpec((PB, 2 * DIM), lambda i: (i, 0)),
        out_shape=jax.ShapeDtypeStruct((n, 2 * DIM), jnp.float32),
    )(ph_t)


def _sc_gather(idx_list, table, total, ch):
    nrole = len(idx_list)
    b_per_w = ch * IW
    mesh = plsc.VectorSubcoreMesh(core_axis_name="c", subcore_axis_name="s")
    row_t = jax.ShapeDtypeStruct((total, 2 * DIM), jnp.float32)

    @functools.partial(
        pl.kernel, mesh=mesh,
        out_type=[row_t] * nrole,
        scratch_types=[pltpu.VMEM((ch, IW), jnp.int32)] * nrole
        + [pltpu.VMEM((2 * IW, 2 * DIM), jnp.float32)] * 2
        + [pltpu.SemaphoreType.DMA] * 2,
    )
    def k(*refs):
        idx_hbm = refs[:nrole]
        table_hbm = refs[nrole]
        outs = refs[nrole + 1:2 * nrole + 1]
        idx_v = refs[2 * nrole + 1:3 * nrole + 1]
        buf0, buf1, sem_g, sem_w = refs[3 * nrole + 1:]
        wid = lax.axis_index("s") * NC + lax.axis_index("c")
        base = wid * b_per_w
        for r in range(nrole):
            pltpu.sync_copy(idx_hbm[r].at[wid], idx_v[r])

        bufs = (buf0, buf1)
        writes = [None, None]
        step = 0
        for r in range(nrole):
            for h in range(ch // 2):
                b = step % 2
                if writes[b] is not None:
                    writes[b].wait()
                g0 = pltpu.async_copy(table_hbm.at[idx_v[r].at[2 * h]],
                                      bufs[b].at[pl.ds(0, IW)], sem_g)
                g1 = pltpu.async_copy(table_hbm.at[idx_v[r].at[2 * h + 1]],
                                      bufs[b].at[pl.ds(IW, IW)], sem_g)
                g0.wait()
                g1.wait()
                writes[b] = pltpu.async_copy(
                    bufs[b], outs[r].at[pl.ds(base + h * 2 * IW, 2 * IW)],
                    sem_w)
                step += 1
        writes[0].wait()
        writes[1].wait()

    return k(*idx_list, table)


def _tc_loss_sum(g_h, g_t, g_r, rows):
    g = rows // (2 * HW)

    sin_c = (0.9999999378197463, -0.16666621108235025, 0.008332791502704946,
             -0.00019817630987702638, 2.70883115859738e-06,
             -2.0698134650665168e-08)
    cos_c = (0.9999992107795053, -0.4999942133837966, 0.041659777806388416,
             -0.0013858789919373926, 2.4202941365944475e-05,
             -2.1972963820671154e-07)

    def body(gh, gt, gr, out):
        ones = jnp.ones((2 * DIM, 2 * DIM), jnp.float32)
        mask = lax.broadcasted_iota(jnp.int32, (2 * HW, 2 * DIM), 1) < DIM
        mrow = lax.broadcasted_iota(jnp.int32, (1, 2 * DIM), 1) < DIM
        coef = [jnp.where(mrow, c, s).astype(jnp.float32)
                for c, s in zip(cos_c, sin_c)]

        def swap(x):
            return jnp.roll(x, DIM, axis=1)

        a = gh[...]
        t = gt[...]
        r = gr[...]
        y = r * r
        p = coef[5]
        for k in (4, 3, 2, 1, 0):
            p = p * y + coef[k]
        cs = jnp.where(mask, p, p * r)
        u = a * cs
        v = a * swap(cs)
        dre2 = u - swap(u)
        dim2 = v + swap(v)
        rot = jnp.where(mask, dre2, dim2)
        diff = rot - t
        sq = diff * diff
        val = jnp.sqrt(sq + swap(sq) + 1e-9)
        mag = jax.lax.dot_general(
            val, ones, (((1,), (0,)), ((), ())),
            preferred_element_type=jnp.float32)
        ms = jnp.maximum(MARGIN + 0.5 * (mag[:HW] - mag[HW:]), 0.0)
        i = pl.program_id(0)

        @pl.when(i == 0)
        def _():
            out[...] = jnp.zeros((1, 1), jnp.float32)

        out[...] += (jnp.sum(ms) / (2 * DIM)).reshape(1, 1)

    spec = pl.BlockSpec((2 * HW, 2 * DIM), lambda i: (i, 0))
    return pl.pallas_call(
        body,
        grid=(g,),
        in_specs=[spec] * 3,
        out_specs=pl.BlockSpec((1, 1), lambda i: (0, 0)),
        out_shape=jax.ShapeDtypeStruct((1, 1), jnp.float32),
    )(g_h, g_t, g_r)


def kernel(positive_triples, negative_triples, entity_re, entity_im,
           relation_phase):
    batch = positive_triples.shape[0]
    cb = batch // NCHUNK
    rows = 2 * cb
    ch = rows // (NW * IW)
    nh = cb // HW
    pt = positive_triples.astype(jnp.int32)
    nt = negative_triples.astype(jnp.int32)

    def order(col_p, col_n):
        mixed = jnp.concatenate([col_p.reshape(nh, HW),
                                 col_n.reshape(nh, HW)], axis=1)
        return mixed.reshape(NW, ch, IW)

    idx = []
    for c in range(NCHUNK):
        sl = slice(c * cb, (c + 1) * cb)
        idx.append([order(pt[sl, k], nt[sl, k]) for k in range(3)])

    e2 = _tc_pack_e2(entity_re.T, entity_im.T)
    ht = [_sc_gather([idx[c][0], idx[c][2]], e2, rows, ch)
          for c in range(NCHUNK)]
    p2 = _tc_pack_p2(relation_phase.T)
    rr = [_sc_gather([idx[c][1]], p2, rows, ch) for c in range(NCHUNK)]
    parts = [_tc_loss_sum(ht[c][0], ht[c][1], rr[c][0], rows)
             for c in range(NCHUNK)]
    total = parts[0]
    for p in parts[1:]:
        total = total + p
    return total[0, 0] / batch
